# baseline (speedup 1.0000x reference)
import functools
import jax
import jax.numpy as jnp
from jax import lax
from jax.experimental import pallas as pl
from jax.experimental.pallas import tpu as pltpu
from jax.experimental.pallas import tpu_sc as plsc

B_, L_, DIM_ = 4, 4096, 1024
NC, NS, LANES = 2, 16, 16
NW = NC * NS
L_PER_W = L_ // NW
R_ = 16
B_TC = 3
BLK_L = 2048


def _tc_body(x_ref, emb_ref, o_ref):
    o_ref[...] = x_ref[...] + emb_ref[...][None, :, :]


def _tc_add(x, emb):
    nb = x.shape[0]
    return pl.pallas_call(
        _tc_body,
        grid=(L_ // BLK_L, nb),
        in_specs=[
            pl.BlockSpec((1, BLK_L, DIM_), lambda i, j: (j, i, 0)),
            pl.BlockSpec((BLK_L, DIM_), lambda i, j: (i, 0)),
        ],
        out_specs=pl.BlockSpec((1, BLK_L, DIM_), lambda i, j: (j, i, 0)),
        out_shape=jax.ShapeDtypeStruct((nb, L_, DIM_), x.dtype),
    )(x, emb)


def _sc_body(nb, x_hbm, emb_hbm, out_hbm, emb_v, xb_v):
    wid = lax.axis_index("s") * NC + lax.axis_index("c")
    nchunks = L_PER_W // R_

    def chunk(c, carry):
        base = wid * L_PER_W + c * R_
        pltpu.sync_copy(emb_hbm.at[pl.ds(base, R_)], emb_v)
        for b in range(nb):
            pltpu.sync_copy(x_hbm.at[b, pl.ds(base, R_)], xb_v.at[b])

        def rbody(r, rcarry):
            for j in range(DIM_ // LANES):
                sl = pl.ds(j * LANES, LANES)
                v = emb_v[r, sl]
                for b in range(nb):
                    plsc.addupdate(xb_v.at[b, r, sl], v)
            return rcarry

        lax.fori_loop(0, R_, rbody, 0)
        for b in range(nb):
            pltpu.sync_copy(xb_v.at[b], out_hbm.at[b, pl.ds(base, R_)])
        return carry

    lax.fori_loop(0, nchunks, chunk, 0)


def _sc_add(x, emb):
    nb = x.shape[0]
    mesh = plsc.VectorSubcoreMesh(core_axis_name="c", subcore_axis_name="s")
    return pl.kernel(
        functools.partial(_sc_body, nb),
        out_type=jax.ShapeDtypeStruct((nb, L_, DIM_), jnp.float32),
        mesh=mesh,
        scratch_types=[
            pltpu.VMEM((R_, DIM_), jnp.float32),
            pltpu.VMEM((nb, R_, DIM_), jnp.float32),
        ],
    )(x, emb)


def kernel(x, emb):
    out_tc = _tc_add(x[:B_TC], emb)
    out_sc = _sc_add(x[B_TC:], emb)
    return jnp.concatenate([out_tc, out_sc], axis=0)


# speedup vs baseline: 1.8287x; 1.8287x over previous
import functools
import jax
import jax.numpy as jnp
from jax import lax
from jax.experimental import pallas as pl
from jax.experimental.pallas import tpu as pltpu
from jax.experimental.pallas import tpu_sc as plsc

B_, L_, DIM_ = 4, 4096, 1024
NC, NS, LANES = 2, 16, 16
NW = NC * NS
L_PER_W = L_ // NW
R_ = 8
NCH = L_PER_W // R_


def _sc_body(x_hbm, emb_hbm, out_hbm, emb_v, xb_v, si0, si1, so0, so1):
    wid = lax.axis_index("s") * NC + lax.axis_index("c")
    row0 = wid * L_PER_W
    sin = (si0, si1)
    sout = (so0, so1)

    def in_copies(slot, c):
        base = row0 + c * R_
        yield pltpu.make_async_copy(emb_hbm.at[pl.ds(base, R_)],
                                    emb_v.at[slot], sin[slot])
        for b in range(B_):
            yield pltpu.make_async_copy(x_hbm.at[b, pl.ds(base, R_)],
                                        xb_v.at[slot, b], sin[slot])

    def out_copies(slot, c):
        base = row0 + c * R_
        for b in range(B_):
            yield pltpu.make_async_copy(xb_v.at[slot, b],
                                        out_hbm.at[b, pl.ds(base, R_)],
                                        sout[slot])

    def compute(slot):
        eslot = emb_v.at[slot]
        xslot = xb_v.at[slot]

        def rbody(r, rcarry):
            for j in range(DIM_ // LANES):
                sl = pl.ds(j * LANES, LANES)
                v = eslot[r, sl]
                for b in range(B_):
                    plsc.addupdate(xslot.at[b, r, sl], v)
            return rcarry

        lax.fori_loop(0, R_, rbody, 0)

    def step(slot, c, first, last):
        if not first:
            for d in out_copies(1 - slot, c - 1):
                d.wait()
        if not last:
            for d in in_copies(1 - slot, c + 1):
                d.start()
        for d in in_copies(slot, c):
            d.wait()
        compute(slot)
        for d in out_copies(slot, c):
            d.start()

    for d in in_copies(0, 0):
        d.start()
    step(0, 0, first=True, last=False)

    @pl.loop(0, (NCH - 2) // 2)
    def _steady(it):
        for k in range(2):
            c = 1 + it * 2 + k
            step((1 + k) % 2, c, first=False, last=False)

    step(1, NCH - 1, first=False, last=True)
    for d in out_copies(1, NCH - 1):
        d.wait()


def kernel(x, emb):
    mesh = plsc.VectorSubcoreMesh(core_axis_name="c", subcore_axis_name="s")
    return pl.kernel(
        _sc_body,
        out_type=jax.ShapeDtypeStruct((B_, L_, DIM_), jnp.float32),
        mesh=mesh,
        scratch_types=[
            pltpu.VMEM((2, R_, DIM_), jnp.float32),
            pltpu.VMEM((2, B_, R_, DIM_), jnp.float32),
            pltpu.SemaphoreType.DMA,
            pltpu.SemaphoreType.DMA,
            pltpu.SemaphoreType.DMA,
            pltpu.SemaphoreType.DMA,
        ],
    )(x, emb)


# speedup vs baseline: 3.2872x; 1.7975x over previous
import jax
import jax.numpy as jnp
from jax.experimental import pallas as pl
from jax.experimental.pallas import tpu as pltpu

DIM_ = 1024
BLK_L = 2048


def _body(x_ref, emb_ref, o_ref):
    o_ref[...] = x_ref[...] + emb_ref[...][None, :, :]


def kernel(x, emb):
    B, L, D = x.shape
    grid = (L // BLK_L, B)
    return pl.pallas_call(
        _body,
        grid=grid,
        in_specs=[
            pl.BlockSpec((1, BLK_L, D), lambda i, j: (j, i, 0)),
            pl.BlockSpec((BLK_L, D), lambda i, j: (i, 0)),
        ],
        out_specs=pl.BlockSpec((1, BLK_L, D), lambda i, j: (j, i, 0)),
        out_shape=jax.ShapeDtypeStruct((B, L, D), x.dtype),
    )(x, emb)
